# Initial kernel scaffold; baseline (speedup 1.0000x reference)
import jax
import jax.numpy as jnp
from jax.experimental import pallas as pl


def kernel(x, edge_index, edge_attr, Wf, bf, bng, bnb, conv_params, fcW, fcb):
    raise NotImplementedError("write your pallas kernel here")


# speedup vs baseline: 3.0933x; 3.0933x over previous
import jax
import jax.numpy as jnp
from jax import lax
from jax.experimental import pallas as pl
from jax.experimental.pallas import tpu as pltpu
from jax.experimental.pallas import tpu_sc as plsc

N = 50000
E = 800000
C = 32
CHUNK = 128
NCORE = 2
NSUB = 16
NW = NCORE * NSUB
CPW = 200
EPAD = NW * CPW * CHUNK
NPAD = EPAD - E
RPTA = 3128
RPT_MAIN = N - (NSUB - 1) * RPTA
GRP = 8

_MESH = plsc.VectorSubcoreMesh(
    core_axis_name="c", subcore_axis_name="s",
    num_cores=NCORE, num_subcores=NSUB)

_f32 = jnp.float32


def _worker():
    cc = lax.axis_index("c")
    ss = lax.axis_index("s")
    wid = ss * NCORE + cc
    return cc, ss, wid


def _striped(ss, fn):
    base = ss * RPTA
    fn(base, RPT_MAIN)

    @pl.when(ss < NSUB - 1)
    def _():
        fn(base + RPT_MAIN, RPTA - RPT_MAIN)


def _counts_body(dst3, src3, ones8, zeros8, out, dbuf, sbuf, onesb, cd_spm, cs_spm):
    cc, ss, wid = _worker()
    _striped(ss, lambda b, n: pltpu.sync_copy(zeros8.at[pl.ds(b, n)],
                                              cd_spm.at[pl.ds(b, n)]))
    _striped(ss, lambda b, n: pltpu.sync_copy(zeros8.at[pl.ds(b, n)],
                                              cs_spm.at[pl.ds(b, n)]))
    pltpu.sync_copy(dst3.at[wid], dbuf)
    pltpu.sync_copy(src3.at[wid], sbuf)
    pltpu.sync_copy(ones8, onesb)
    plsc.subcore_barrier()

    def body(j, carry):
        pltpu.sync_copy(onesb, cd_spm.at[dbuf.at[j]], add=True)
        pltpu.sync_copy(onesb, cs_spm.at[sbuf.at[j]], add=True)
        return carry

    lax.fori_loop(0, CPW, body, 0)

    plsc.subcore_barrier()
    _striped(ss, lambda b, n: pltpu.sync_copy(cd_spm.at[pl.ds(b, n)],
                                              out.at[cc, 0, pl.ds(b, n)]))
    _striped(ss, lambda b, n: pltpu.sync_copy(cs_spm.at[pl.ds(b, n)],
                                              out.at[cc, 1, pl.ds(b, n)]))


_counts_call = pl.kernel(
    _counts_body,
    out_type=jax.ShapeDtypeStruct((NCORE, 2, N, 8), _f32),
    mesh=_MESH,
    compiler_params=pltpu.CompilerParams(use_tc_tiling_on_sc=False),
    scratch_types=[
        pltpu.VMEM((CPW, CHUNK), jnp.int32),
        pltpu.VMEM((CPW, CHUNK), jnp.int32),
        pltpu.VMEM((CHUNK, 8), _f32),
        pltpu.VMEM_SHARED((N, 8), _f32),
        pltpu.VMEM_SHARED((N, 8), _f32),
    ],
)


def _cross_body(ad, bs, dst3, src3, out, dbuf, sbuf, arows, brows, obuf):
    cc, ss, wid = _worker()
    pltpu.sync_copy(dst3.at[wid], dbuf)
    pltpu.sync_copy(src3.at[wid], sbuf)

    def chunk_acc(j, carry):
        clo, chi = carry
        pltpu.sync_copy(ad.at[dbuf.at[j]], arows)
        pltpu.sync_copy(bs.at[sbuf.at[j]], brows)

        def row(j2, cr):
            cl, ch = cr
            cl = cl + arows[j2, pl.ds(0, 16)] * brows[j2, pl.ds(0, 16)]
            ch = ch + arows[j2, pl.ds(16, 16)] * brows[j2, pl.ds(16, 16)]
            return (cl, ch)

        return lax.fori_loop(0, CHUNK, row, (clo, chi), unroll=2)

    z = jnp.zeros((16,), _f32)
    clo, chi = lax.fori_loop(0, CPW, chunk_acc, (z, z))
    obuf[pl.ds(0, 16)] = clo
    obuf[pl.ds(16, 16)] = chi
    pltpu.sync_copy(obuf, out.at[wid])


_cross_call = pl.kernel(
    _cross_body,
    out_type=jax.ShapeDtypeStruct((NW, C), _f32),
    mesh=_MESH,
    compiler_params=pltpu.CompilerParams(use_tc_tiling_on_sc=False),
    scratch_types=[
        pltpu.VMEM((CPW, CHUNK), jnp.int32),
        pltpu.VMEM((CPW, CHUNK), jnp.int32),
        pltpu.VMEM((CHUNK, C), _f32),
        pltpu.VMEM((CHUNK, C), _f32),
        pltpu.VMEM((C,), _f32),
    ],
)


def _edge_main_body(ad2, bs2, dst3, src3, ea3, zeros32, out,
                    dbuf, sbuf, eabuf, arows, brows, spm):
    cc, ss, wid = _worker()
    _striped(ss, lambda b, n: pltpu.sync_copy(zeros32.at[pl.ds(b, n)],
                                              spm.at[pl.ds(b, n)]))
    plsc.subcore_barrier()

    def do_group(go, carry):
        pltpu.sync_copy(dst3.at[wid, pl.ds(go * GRP, GRP)], dbuf)
        pltpu.sync_copy(src3.at[wid, pl.ds(go * GRP, GRP)], sbuf)
        pltpu.sync_copy(ea3.at[wid, pl.ds(go * GRP, GRP)], eabuf)

        def do_chunk(j, c1):
            pltpu.sync_copy(ad2.at[dbuf.at[j]], arows)
            pltpu.sync_copy(bs2.at[sbuf.at[j]], brows)

            def grp(g, c2):
                base = g * 16
                ea16 = eabuf[j, pl.ds(base, 16)]
                for i in range(16):
                    j2 = base + i
                    e = ea16[i]
                    alo = arows[j2, pl.ds(0, 16)]
                    ahi = arows[j2, pl.ds(16, 16)]
                    blo = brows[j2, pl.ds(0, 16)]
                    bhi = brows[j2, pl.ds(16, 16)]
                    arows[j2, pl.ds(0, 16)] = jnp.maximum(alo + blo, 0.0) * e
                    arows[j2, pl.ds(16, 16)] = jnp.maximum(ahi + bhi, 0.0) * e
                return c2

            lax.fori_loop(0, CHUNK // 16, grp, 0)
            pltpu.sync_copy(arows, spm.at[dbuf.at[j]], add=True)
            return c1

        lax.fori_loop(0, GRP, do_chunk, 0)
        return carry

    lax.fori_loop(0, CPW // GRP, do_group, 0)

    plsc.subcore_barrier()
    _striped(ss, lambda b, n: pltpu.sync_copy(spm.at[pl.ds(b, n)],
                                              out.at[cc, pl.ds(b, n)]))


_edge_main_call = pl.kernel(
    _edge_main_body,
    out_type=jax.ShapeDtypeStruct((NCORE, N, C), _f32),
    mesh=_MESH,
    compiler_params=pltpu.CompilerParams(use_tc_tiling_on_sc=False),
    scratch_types=[
        pltpu.VMEM((GRP, CHUNK), jnp.int32),
        pltpu.VMEM((GRP, CHUNK), jnp.int32),
        pltpu.VMEM((GRP, CHUNK), _f32),
        pltpu.VMEM((CHUNK, C), _f32),
        pltpu.VMEM((CHUNK, C), _f32),
        pltpu.VMEM_SHARED((N, C), _f32),
    ],
)


_BP = 2000


def _prologue_body(xa, xv, xaf, xvf, wfa, wfv, bfb, o_ref):
    af = xaf[...]
    vf = xvf[...]
    dot = jnp.sum(af * vf, axis=1)
    na = jnp.sqrt(jnp.sum(af * af, axis=1))
    nv = jnp.sqrt(jnp.sum(vf * vf, axis=1))
    sim = dot / jnp.maximum(na * nv, 1e-8)
    g = (jnp.dot(xa[...] * sim[:, None], wfa[...], preferred_element_type=_f32)
         + jnp.dot(xv[...], wfv[...], preferred_element_type=_f32) + bfb[...])
    o_ref[...] = g


_prologue_call = pl.pallas_call(
    _prologue_body,
    out_shape=jax.ShapeDtypeStruct((N, C), _f32),
    grid=(N // _BP,),
    in_specs=[pl.BlockSpec((_BP, 128), lambda i: (i, 0))] * 4
    + [pl.BlockSpec((128, C), lambda i: (0, 0))] * 2
    + [pl.BlockSpec((1, C), lambda i: (0, 0))],
    out_specs=pl.BlockSpec((_BP, C), lambda i: (i, 0)),
)


def _bn_n(y, g, b, eps=1e-5):
    m = jnp.mean(y, axis=0, keepdims=True)
    v = jnp.var(y, axis=0, keepdims=True)
    return (y - m) / jnp.sqrt(v + eps) * g + b


def kernel(x, edge_index, edge_attr, Wf, bf, bng, bnb, conv_params, fcW, fcb):
    src = edge_index[0]
    dst = edge_index[1]
    ipad = jnp.zeros((NPAD,), dst.dtype)
    dst3 = jnp.concatenate([dst, ipad]).reshape(NW, CPW, CHUNK)
    src3 = jnp.concatenate([src, ipad]).reshape(NW, CPW, CHUNK)
    ea3 = jnp.concatenate([edge_attr[:, 0], jnp.zeros((NPAD,), _f32)]
                          ).reshape(NW, CPW, CHUNK)
    zeros32 = jnp.zeros((N, C), _f32)
    zeros8 = jnp.zeros((N, 8), _f32)
    ones8 = jnp.ones((CHUNK, 8), _f32)

    cnts = _counts_call(dst3, src3, ones8, zeros8)
    cnt_d = (cnts[0, 0] + cnts[1, 0])[:, 0]
    cnt_s = (cnts[0, 1] + cnts[1, 1])[:, 0]
    npadv = jnp.zeros((N,), _f32).at[0].set(_f32(NPAD))
    cnt_d = cnt_d - npadv
    cnt_s = cnt_s - npadv

    g_pre = _prologue_call(x[:, 0, :], x[:, 1, :], x[:, 2, :], x[:, 3, :],
                           Wf[:, :128].T, Wf[:, 128:].T, bf[None, :])
    h = jax.nn.relu(_bn_n(g_pre, bng[0], bnb[0]))

    Ef = _f32(E)
    for k in range(4):
        g1, b1, W1, g2, b2, W2 = conv_params[k]
        hh = h * h
        mu_d = (cnt_d @ h) / Ef
        var_d = (cnt_d @ hh) / Ef - mu_d ** 2
        mu_s = (cnt_s @ h) / Ef
        var_s = (cnt_s @ hh) / Ef - mu_s ** 2
        sc_d = g1[:C] / jnp.sqrt(var_d + 1e-5)
        sh_d = b1[:C] - mu_d * sc_d
        sc_s = g1[C:] / jnp.sqrt(var_s + 1e-5)
        sh_s = b1[C:] - mu_s * sc_s
        Ad = jax.nn.relu(h * sc_d + sh_d) @ W1[:, :C].T
        Bs = jax.nn.relu(h * sc_s + sh_s) @ W1[:, C:].T

        cross = jnp.sum(_cross_call(Ad, Bs, dst3, src3), axis=0)
        cross = cross - _f32(NPAD) * Ad[0] * Bs[0]

        mu2 = (cnt_d @ Ad + cnt_s @ Bs) / Ef
        ex2 = (cnt_d @ (Ad * Ad) + cnt_s @ (Bs * Bs) + 2.0 * cross) / Ef
        var2 = ex2 - mu2 ** 2
        sc2 = g2 / jnp.sqrt(var2 + 1e-5)
        sh2 = b2 - mu2 * sc2
        Ad2 = Ad * sc2 + sh2
        Bs2 = Bs * sc2

        Sp = _edge_main_call(Ad2, Bs2, dst3, src3, ea3, zeros32)
        S = Sp[0] + Sp[1]
        conv = (S / jnp.maximum(cnt_d, 1.0)[:, None]) @ W2.T
        if k < 3:
            hnew = conv + h if k > 0 else conv
            h = jax.nn.relu(_bn_n(hnew, bng[k + 1], bnb[k + 1]))
        else:
            h = conv + h
    return h @ fcW.T + fcb
